# Initial kernel scaffold; baseline (speedup 1.0000x reference)
import jax
import jax.numpy as jnp
from jax.experimental import pallas as pl


def kernel(pointcloud, numpoints, params):
    raise NotImplementedError("write your pallas kernel here")


# speedup vs baseline: 9.4346x; 9.4346x over previous
import functools

import jax
import jax.numpy as jnp
from jax import lax
from jax.experimental import pallas as pl
from jax.experimental.pallas import tpu as pltpu
from jax.experimental.pallas import tpu_sc as plsc

_B = 8
_NSAMP = 32
_BN_EPS = 1e-5
_NC, _NSUB, _L = 2, 16, 16
_NW = _NC * _NSUB


def _sc_mesh():
    return plsc.VectorSubcoreMesh(
        core_axis_name="c", subcore_axis_name="s", num_cores=_NC, num_subcores=_NSUB
    )


def _make_ball_query(N, S, radius):
    Q = _B * S
    qpw = Q // _NW
    nchunk = N // _L
    r2 = radius * radius

    @functools.partial(
        pl.kernel,
        out_type=(
            jax.ShapeDtypeStruct((Q * _NSAMP * 3,), jnp.float32),
            jax.ShapeDtypeStruct((Q * _NSAMP,), jnp.int32),
        ),
        mesh=_sc_mesh(),
        scratch_types=[
            pltpu.VMEM((N,), jnp.float32),
            pltpu.VMEM((N,), jnp.float32),
            pltpu.VMEM((N,), jnp.float32),
            pltpu.VMEM((N,), jnp.float32),
            pltpu.VMEM((N,), jnp.float32),
            pltpu.VMEM((N,), jnp.float32),
            pltpu.VMEM((N,), jnp.float32),
            pltpu.VMEM((_NSAMP + _L,), jnp.int32),
            pltpu.VMEM((qpw * _NSAMP * 3,), jnp.float32),
            pltpu.VMEM((qpw * _NSAMP,), jnp.int32),
        ],
        compiler_params=pltpu.CompilerParams(needs_layout_passes=False),
    )
    def kern(xyz_hbm, oxyz_hbm, oidx_hbm, px, py, pz, pp, pxr, pyr, pzr, ibuf, obuf, gibuf):
        wid = lax.axis_index("s") * _NC + lax.axis_index("c")
        q0 = wid * qpw
        b = q0 // S
        s0 = q0 - b * S
        pltpu.sync_copy(xyz_hbm.at[pl.ds((b * 3 + 0) * N, N)], px)
        pltpu.sync_copy(xyz_hbm.at[pl.ds((b * 3 + 1) * N, N)], py)
        pltpu.sync_copy(xyz_hbm.at[pl.ds((b * 3 + 2) * N, N)], pz)

        def _bf16_round(v):
            u = plsc.bitcast(v, jnp.uint32)
            r = (u + jnp.uint32(0x7FFF) + ((u >> jnp.uint32(16)) & jnp.uint32(1))) & jnp.uint32(
                0xFFFF0000
            )
            return plsc.bitcast(r, jnp.float32)

        def norm_body(i, carry):
            sl = pl.ds(i * _L, _L)
            x = px[sl]
            y = py[sl]
            z = pz[sl]
            pp[sl] = x * x + y * y + z * z
            pxr[sl] = _bf16_round(x)
            pyr[sl] = _bf16_round(y)
            pzr[sl] = _bf16_round(z)
            return carry

        lax.fori_loop(0, nchunk, norm_body, 0)
        lane = lax.iota(jnp.int32, _L)

        def q_body(j, carry):
            s = s0 + j
            sv = jnp.broadcast_to(s, (_L,))
            qx = plsc.load_gather(px, [sv])
            qy = plsc.load_gather(py, [sv])
            qz = plsc.load_gather(pz, [sv])
            qq = plsc.load_gather(pp, [sv])
            qxr = plsc.load_gather(pxr, [sv])
            qyr = plsc.load_gather(pyr, [sv])
            qzr = plsc.load_gather(pzr, [sv])

            def cond(c):
                i, cnt = c
                return jnp.logical_and(i < nchunk, cnt < _NSAMP)

            def body(c):
                i, cnt = c
                sl = pl.ds(i * _L, _L)
                d2 = qq + pp[sl] - 2.0 * (qxr * pxr[sl] + qyr * pyr[sl] + qzr * pzr[sl])
                m = d2 < r2
                plsc.store_compressed(ibuf.at[pl.ds(cnt, _L)], i * _L + lane, mask=m)
                return i + 1, cnt + jnp.sum(m.astype(jnp.int32))

            _, cnt = lax.while_loop(cond, body, (jnp.int32(0), jnp.int32(0)))
            first = ibuf[pl.ds(0, _L)][0]
            for v in range(_NSAMP // _L):
                iv = ibuf[pl.ds(v * _L, _L)]
                iv = jnp.where(lane + (v * _L) < cnt, iv, first)
                gx = plsc.load_gather(px, [iv]) - qx
                gy = plsc.load_gather(py, [iv]) - qy
                gz = plsc.load_gather(pz, [iv]) - qz
                pos = (j * _NSAMP + v * _L + lane) * 3
                plsc.store_scatter(obuf, [pos], gx)
                plsc.store_scatter(obuf, [pos + 1], gy)
                plsc.store_scatter(obuf, [pos + 2], gz)
                gibuf[pl.ds(j * _NSAMP + v * _L, _L)] = iv + b * N
            return carry

        lax.fori_loop(0, qpw, q_body, 0)
        pltpu.sync_copy(obuf, oxyz_hbm.at[pl.ds(q0 * _NSAMP * 3, qpw * _NSAMP * 3)])
        pltpu.sync_copy(gibuf, oidx_hbm.at[pl.ds(q0 * _NSAMP, qpw * _NSAMP)])

    return kern


def _make_feat_gather(BN, C, QN):
    R = QN // _NW
    CH = 128
    nch = R // CH

    @functools.partial(
        pl.kernel,
        out_type=jax.ShapeDtypeStruct((QN, C), jnp.float32),
        mesh=_sc_mesh(),
        scratch_types=[
            pltpu.VMEM((R,), jnp.int32),
            pltpu.VMEM((CH, C), jnp.float32),
            pltpu.VMEM((CH, C), jnp.float32),
            pltpu.SemaphoreType.DMA,
            pltpu.SemaphoreType.DMA,
        ],
        compiler_params=pltpu.CompilerParams(needs_layout_passes=False),
    )
    def kern(tab_hbm, rows_hbm, out_hbm, idxv, buf0, buf1, sem0, sem1):
        wid = lax.axis_index("s") * _NC + lax.axis_index("c")
        r0 = wid * R
        pltpu.sync_copy(rows_hbm.at[pl.ds(r0, R)], idxv)
        bufs = (buf0, buf1)
        sems = (sem0, sem1)
        descs = [None, None]
        descs[0] = pltpu.async_copy(
            tab_hbm.at[idxv.at[pl.ds(0, CH)]], buf0, sem0
        )
        for i in range(nch):
            cur = i % 2
            nxt = i + 1
            if nxt < nch:
                descs[nxt % 2] = pltpu.async_copy(
                    tab_hbm.at[idxv.at[pl.ds(nxt * CH, CH)]], bufs[nxt % 2], sems[nxt % 2]
                )
            descs[cur].wait()
            pltpu.sync_copy(bufs[cur], out_hbm.at[pl.ds(r0 + i * CH, CH)])

    return kern


def _stats_update(st_ref, y, is_first):
    @pl.when(is_first)
    def _():
        st_ref[...] = jnp.zeros_like(st_ref)

    st_ref[0:1, :] += jnp.sum(y, axis=0, keepdims=True)
    st_ref[1:2, :] += jnp.sum(y * y, axis=0, keepdims=True)


def _bn_coeffs(st, g, bt, pf):
    mean = st[0:1, :] * (1.0 / pf)
    var = st[1:2, :] * (1.0 / pf) - mean * mean
    rinv = lax.rsqrt(var + _BN_EPS)
    scale = rinv * g
    shift = bt - mean * scale
    return scale, shift


def _mm_entry_xyz(xxyz, w, b, Pt=2048):
    P, _ = xxyz.shape
    Cout = w.shape[1]
    grid = P // Pt

    def body(x_ref, w_ref, b_ref, y_ref, st_ref):
        y = jnp.dot(x_ref[...], w_ref[...], preferred_element_type=jnp.float32)
        y = y + b_ref[...]
        y_ref[...] = y
        _stats_update(st_ref, y, pl.program_id(0) == 0)

    return pl.pallas_call(
        body,
        grid=(grid,),
        in_specs=[
            pl.BlockSpec((Pt, 3), lambda i: (i, 0)),
            pl.BlockSpec((3, Cout), lambda i: (0, 0)),
            pl.BlockSpec((1, Cout), lambda i: (0, 0)),
        ],
        out_specs=[
            pl.BlockSpec((Pt, Cout), lambda i: (i, 0)),
            pl.BlockSpec((8, Cout), lambda i: (0, 0)),
        ],
        out_shape=[
            jax.ShapeDtypeStruct((P, Cout), jnp.float32),
            jax.ShapeDtypeStruct((8, Cout), jnp.float32),
        ],
    )(xxyz, w, b)


def _mm_entry_concat(xxyz, xfeat, wx, wf, b, Pt=1024):
    P, _ = xxyz.shape
    Cin = xfeat.shape[1]
    Cout = wx.shape[1]
    grid = P // Pt

    def body(x_ref, f_ref, wx_ref, wf_ref, b_ref, y_ref, st_ref):
        y = jnp.dot(f_ref[...], wf_ref[...], preferred_element_type=jnp.float32)
        y = y + jnp.dot(x_ref[...], wx_ref[...], preferred_element_type=jnp.float32)
        y = y + b_ref[...]
        y_ref[...] = y
        _stats_update(st_ref, y, pl.program_id(0) == 0)

    return pl.pallas_call(
        body,
        grid=(grid,),
        in_specs=[
            pl.BlockSpec((Pt, 3), lambda i: (i, 0)),
            pl.BlockSpec((Pt, Cin), lambda i: (i, 0)),
            pl.BlockSpec((3, Cout), lambda i: (0, 0)),
            pl.BlockSpec((Cin, Cout), lambda i: (0, 0)),
            pl.BlockSpec((1, Cout), lambda i: (0, 0)),
        ],
        out_specs=[
            pl.BlockSpec((Pt, Cout), lambda i: (i, 0)),
            pl.BlockSpec((8, Cout), lambda i: (0, 0)),
        ],
        out_shape=[
            jax.ShapeDtypeStruct((P, Cout), jnp.float32),
            jax.ShapeDtypeStruct((8, Cout), jnp.float32),
        ],
    )(xxyz, xfeat, wx, wf, b)


def _mm_mid(yprev, st, g, bt, w, b, Pt=1024):
    P, Cin = yprev.shape
    Cout = w.shape[1]
    grid = P // Pt
    pf = float(P)

    def body(yp_ref, st_in_ref, g_ref, bt_ref, w_ref, b_ref, y_ref, st_ref):
        scale, shift = _bn_coeffs(st_in_ref[...], g_ref[...], bt_ref[...], pf)
        x = jnp.maximum(yp_ref[...] * scale + shift, 0.0)
        y = jnp.dot(x, w_ref[...], preferred_element_type=jnp.float32) + b_ref[...]
        y_ref[...] = y
        _stats_update(st_ref, y, pl.program_id(0) == 0)

    return pl.pallas_call(
        body,
        grid=(grid,),
        in_specs=[
            pl.BlockSpec((Pt, Cin), lambda i: (i, 0)),
            pl.BlockSpec((8, Cin), lambda i: (0, 0)),
            pl.BlockSpec((1, Cin), lambda i: (0, 0)),
            pl.BlockSpec((1, Cin), lambda i: (0, 0)),
            pl.BlockSpec((Cin, Cout), lambda i: (0, 0)),
            pl.BlockSpec((1, Cout), lambda i: (0, 0)),
        ],
        out_specs=[
            pl.BlockSpec((Pt, Cout), lambda i: (i, 0)),
            pl.BlockSpec((8, Cout), lambda i: (0, 0)),
        ],
        out_shape=[
            jax.ShapeDtypeStruct((P, Cout), jnp.float32),
            jax.ShapeDtypeStruct((8, Cout), jnp.float32),
        ],
    )(yprev, st, g, bt, w, b)


def _bn_relu_maxpool(ylast, st, g, bt, Bt=64):
    BS, ns, C = ylast.shape
    grid = BS // Bt
    pf = float(BS * ns)

    def body(y_ref, st_ref, g_ref, bt_ref, o_ref):
        scale, shift = _bn_coeffs(st_ref[...], g_ref[...], bt_ref[...], pf)
        x = jnp.maximum(y_ref[...] * scale[None, :, :] + shift[None, :, :], 0.0)
        o_ref[...] = jnp.max(x, axis=1)

    return pl.pallas_call(
        body,
        grid=(grid,),
        in_specs=[
            pl.BlockSpec((Bt, ns, C), lambda i: (i, 0, 0)),
            pl.BlockSpec((8, C), lambda i: (0, 0)),
            pl.BlockSpec((1, C), lambda i: (0, 0)),
            pl.BlockSpec((1, C), lambda i: (0, 0)),
        ],
        out_specs=pl.BlockSpec((Bt, C), lambda i: (i, 0)),
        out_shape=jax.ShapeDtypeStruct((BS, C), jnp.float32),
    )(ylast, st, g, bt)


def _prep_layers(layers):
    out = []
    for (W, b, g, bt) in layers:
        out.append((W.T, b[None, :], g[None, :], bt[None, :]))
    return out


def _stage(xyz_soa, feat_rows, N, S, radius, layers):
    gxyz_flat, grows = _make_ball_query(N, S, radius)(xyz_soa.reshape(-1))
    P = _B * S * _NSAMP
    xxyz = gxyz_flat.reshape(P, 3)
    lyr = _prep_layers(layers)
    (w1, b1, g1, t1) = lyr[0]
    if feat_rows is None:
        y, st = _mm_entry_xyz(xxyz, w1, b1)
    else:
        C0 = feat_rows.shape[1]
        gfeat = _make_feat_gather(feat_rows.shape[0], C0, P)(feat_rows, grows)
        y, st = _mm_entry_concat(xxyz, gfeat, w1[:3], w1[3:], b1)
    for (w, b, g, bt) in lyr[1:]:
        y, st2 = _mm_mid(y, st, g1, t1, w, b)
        st, (g1, t1) = st2, (g, bt)
    Cend = y.shape[1]
    pooled = _bn_relu_maxpool(y.reshape(_B * S, _NSAMP, Cend), st, g1, t1)
    return pooled


def kernel(pointcloud, numpoints, params):
    xyz = pointcloud[..., 0:3]
    xyz_soa = jnp.transpose(xyz, (0, 2, 1))
    N0 = xyz.shape[1]
    npts = jnp.asarray(numpoints).astype(jnp.int32)

    f1 = _stage(xyz_soa, None, N0, 1024, 0.3, params[0])
    f2 = _stage(xyz_soa[:, :, :1024], f1, 1024, 256, 0.5, params[1])
    f3 = _stage(xyz_soa[:, :, :256], f2, 256, 64, 0.7, params[2])

    new_xyz3 = xyz[:, :64, :]
    feats3 = jnp.transpose(f3.reshape(_B, 64, -1), (0, 2, 1))
    base = jnp.minimum(jnp.arange(1024, dtype=jnp.int32), npts[0] - 1)
    idx0 = jnp.broadcast_to(base[None, :], (_B, 1024))
    return (new_xyz3, feats3, idx0)
